# baseline (speedup 1.0000x reference)
import functools

import jax
import jax.numpy as jnp
from jax import lax
from jax.experimental import pallas as pl
from jax.experimental.pallas import tpu as pltpu
from jax.experimental.pallas import tpu_sc as plsc

VOCAB = 1000000
EMBED_DIM = 16
BATCH = 16384
_ROWS_PER_BLOCK = 128 // EMBED_DIM
_NUM_BLOCKS = VOCAB // _ROWS_PER_BLOCK

_info = plsc.get_sparse_core_info()
_NC, _NS = _info.num_cores, _info.num_subcores
_NW = _NC * _NS
_B_PER_W = BATCH // _NW
_CHUNK = 128
_N_CHUNKS = _B_PER_W // _CHUNK
_LANES = 16
_OUT_PER_W = _B_PER_W * EMBED_DIM

_mesh = plsc.VectorSubcoreMesh(core_axis_name="c", subcore_axis_name="s")


@functools.partial(
    pl.kernel,
    mesh=_mesh,
    out_type=jax.ShapeDtypeStruct((BATCH * EMBED_DIM,), jnp.float32),
    scratch_types=[
        pltpu.VMEM((_B_PER_W,), jnp.int32),
        pltpu.VMEM((_B_PER_W,), jnp.int32),
        pltpu.VMEM((_B_PER_W, 128), jnp.float32),
        pltpu.VMEM((_OUT_PER_W,), jnp.float32),
        pltpu.SemaphoreType.DMA,
    ],
    compiler_params=pltpu.CompilerParams(needs_layout_passes=False),
)
def _gather_kernel(table_hbm, idx_hbm, out_hbm, idx_v, bid_v, blocks_v,
                   rows_v, sem):
    wid = lax.axis_index("s") * _NC + lax.axis_index("c")
    base = wid * _B_PER_W
    pltpu.sync_copy(idx_hbm.at[pl.ds(base, _B_PER_W)], idx_v)

    def bid_body(j, carry):
        v = idx_v[pl.ds(j * _LANES, _LANES)]
        bid_v[pl.ds(j * _LANES, _LANES)] = lax.shift_right_logical(v, 3)
        return carry

    lax.fori_loop(0, _B_PER_W // _LANES, bid_body, 0)

    copies = []
    for j in range(_N_CHUNKS):
        copies.append(
            pltpu.async_copy(
                table_hbm.at[bid_v.at[pl.ds(j * _CHUNK, _CHUNK)]],
                blocks_v.at[pl.ds(j * _CHUNK, _CHUNK)],
                sem,
            )
        )
    for c in copies:
        c.wait()

    lanes = lax.iota(jnp.int32, _LANES)

    def sel_body(j, carry):
        rows = lanes + j * _LANES
        sub = (idx_v[pl.ds(j * _LANES, _LANES)] & 7) * EMBED_DIM
        out_base = rows * EMBED_DIM
        for col in range(EMBED_DIM):
            vals = plsc.load_gather(blocks_v, [rows, sub + col])
            plsc.store_scatter(rows_v, [out_base + col], vals)
        return carry

    lax.fori_loop(0, _B_PER_W // _LANES, sel_body, 0)

    pltpu.sync_copy(rows_v, out_hbm.at[pl.ds(base * EMBED_DIM, _OUT_PER_W)])


def kernel(g, x, attri):
    idx = jnp.squeeze(x).astype(jnp.int32)
    table = attri.reshape(_NUM_BLOCKS, 128)
    flat = _gather_kernel(table, idx)
    return flat.reshape(BATCH, EMBED_DIM)


# speedup vs baseline: 5.3350x; 5.3350x over previous
import functools

import jax
import jax.numpy as jnp
from jax import lax
from jax.experimental import pallas as pl
from jax.experimental.pallas import tpu as pltpu
from jax.experimental.pallas import tpu_sc as plsc

VOCAB = 1000000
EMBED_DIM = 16
BATCH = 16384

_info = plsc.get_sparse_core_info()
_NC, _NS = _info.num_cores, _info.num_subcores
_NW = _NC * _NS
_B_PER_W = BATCH // _NW
_LANES = 16
_WAVE = 32
_N_WAVES = _B_PER_W // _WAVE
_GROUPS_PER_WAVE = _WAVE // _LANES

_mesh = plsc.VectorSubcoreMesh(core_axis_name="c", subcore_axis_name="s")


@functools.partial(
    pl.kernel,
    mesh=_mesh,
    out_type=jax.ShapeDtypeStruct((EMBED_DIM, BATCH), jnp.float32),
    scratch_types=[
        pltpu.VMEM((_B_PER_W,), jnp.int32),
        pltpu.VMEM((_WAVE, EMBED_DIM, 128), jnp.float32),
        pltpu.VMEM((EMBED_DIM, _B_PER_W), jnp.float32),
        pltpu.SemaphoreType.DMA,
    ],
    compiler_params=pltpu.CompilerParams(needs_layout_passes=False),
)
def _gather_kernel(table_hbm, idx_hbm, out_hbm, idx_v, blocks_v, stage_v,
                   sem):
    wid = lax.axis_index("s") * _NC + lax.axis_index("c")
    base = wid * _B_PER_W
    pltpu.sync_copy(idx_hbm.at[pl.ds(base, _B_PER_W)], idx_v)

    lanes = lax.iota(jnp.int32, _LANES)

    def wave_body(k, carry):
        copies = []
        for gg in range(_GROUPS_PER_WAVE):
            vec = idx_v[pl.ds((k * _GROUPS_PER_WAVE + gg) * _LANES, _LANES)]
            vbase_vec = (vec >> 7) << 7
            for l in range(_LANES):
                vbase = jnp.sum(jnp.where(lanes == l, vbase_vec, 0))
                vbase = pl.multiple_of(vbase, 128)
                copies.append(
                    pltpu.async_copy(
                        table_hbm.at[lanes, pl.ds(vbase, 128)],
                        blocks_v.at[gg * _LANES + l],
                        sem,
                    )
                )
        for cp in copies:
            cp.wait()

        for gg in range(_GROUPS_PER_WAVE):
            vec = idx_v[pl.ds((k * _GROUPS_PER_WAVE + gg) * _LANES, _LANES)]
            cvec = vec & 127
            for l in range(_LANES):
                c = jnp.sum(jnp.where(lanes == l, cvec, 0))
                vals = plsc.load_gather(
                    blocks_v,
                    [
                        jnp.full((_LANES,), gg * _LANES + l, jnp.int32),
                        lanes,
                        jnp.full((_LANES,), c, jnp.int32),
                    ],
                )
                plsc.store_scatter(
                    stage_v,
                    [
                        lanes,
                        jnp.full(
                            (_LANES,), k * _WAVE + gg * _LANES + l, jnp.int32
                        ),
                    ],
                    vals,
                )
        return carry

    lax.fori_loop(0, _N_WAVES, wave_body, 0)

    pltpu.sync_copy(stage_v, out_hbm.at[:, pl.ds(base, _B_PER_W)])


def kernel(g, x, attri):
    idx = jnp.squeeze(x).astype(jnp.int32)
    out_t = _gather_kernel(attri.T, idx)
    return out_t.T


# speedup vs baseline: 6.8765x; 1.2890x over previous
import functools

import jax
import jax.numpy as jnp
from jax import lax
from jax.experimental import pallas as pl
from jax.experimental.pallas import tpu as pltpu
from jax.experimental.pallas import tpu_sc as plsc

VOCAB = 1000000
EMBED_DIM = 16
BATCH = 16384

_info = plsc.get_sparse_core_info()
_NC, _NS = _info.num_cores, _info.num_subcores
_NW = _NC * _NS
_B_PER_W = BATCH // _NW
_LANES = 16
_WAVE = 16
_N_WAVES = _B_PER_W // _WAVE

_mesh = plsc.VectorSubcoreMesh(core_axis_name="c", subcore_axis_name="s")


@functools.partial(
    pl.kernel,
    mesh=_mesh,
    out_type=jax.ShapeDtypeStruct((EMBED_DIM, BATCH), jnp.float32),
    scratch_types=[
        pltpu.VMEM((_B_PER_W,), jnp.int32),
        pltpu.VMEM((2, _WAVE, EMBED_DIM, 128), jnp.float32),
        pltpu.VMEM((EMBED_DIM, _B_PER_W), jnp.float32),
        pltpu.SemaphoreType.DMA((2,)),
    ],
    compiler_params=pltpu.CompilerParams(needs_layout_passes=False),
)
def _gather_kernel(table_hbm, idx_hbm, out_hbm, idx_v, blocks_v, stage_v,
                   sems):
    wid = lax.axis_index("s") * _NC + lax.axis_index("c")
    base = wid * _B_PER_W
    pltpu.sync_copy(idx_hbm.at[pl.ds(base, _B_PER_W)], idx_v)

    lanes = lax.iota(jnp.int32, _LANES)
    zero128 = pl.multiple_of(jnp.int32(0), 128)

    def issue_wave(k, buf):
        vec = idx_v[pl.ds(k * _WAVE, _WAVE)]
        vbase_vec = (vec >> 7) << 7
        for l in range(_WAVE):
            vbase = jnp.sum(jnp.where(lanes == l, vbase_vec, 0))
            vbase = pl.multiple_of(vbase, 128)
            pltpu.async_copy(
                table_hbm.at[lanes, pl.ds(vbase, 128)],
                blocks_v.at[buf, l],
                sems.at[buf],
            )

    def drain_wave(buf):
        for l in range(_WAVE):
            pltpu.make_async_copy(
                table_hbm.at[lanes, pl.ds(zero128, 128)],
                blocks_v.at[buf, l],
                sems.at[buf],
            ).wait()

    def select_wave(k, buf):
        vec = idx_v[pl.ds(k * _WAVE, _WAVE)]
        cvec = vec & 127
        bufvec = jnp.full((_LANES,), buf, jnp.int32)
        for l in range(_WAVE):
            c = jnp.sum(jnp.where(lanes == l, cvec, 0))
            vals = plsc.load_gather(
                blocks_v,
                [
                    bufvec,
                    jnp.full((_LANES,), l, jnp.int32),
                    lanes,
                    jnp.full((_LANES,), c, jnp.int32),
                ],
            )
            plsc.store_scatter(
                stage_v,
                [lanes, jnp.full((_LANES,), k * _WAVE + l, jnp.int32)],
                vals,
            )

    issue_wave(0, jnp.int32(0))

    def wave_body(k, carry):
        buf = lax.rem(k, 2)
        nxt = 1 - buf

        @pl.when(k + 1 < _N_WAVES)
        def _():
            issue_wave(k + 1, nxt)

        drain_wave(buf)
        select_wave(k, buf)
        return carry

    lax.fori_loop(0, _N_WAVES, wave_body, 0)

    pltpu.sync_copy(stage_v, out_hbm.at[:, pl.ds(base, _B_PER_W)])


def kernel(g, x, attri):
    idx = jnp.squeeze(x).astype(jnp.int32)
    out_t = _gather_kernel(attri.T, idx)
    return out_t.T


# speedup vs baseline: 6.9171x; 1.0059x over previous
import functools

import jax
import jax.numpy as jnp
from jax import lax
from jax.experimental import pallas as pl
from jax.experimental.pallas import tpu as pltpu
from jax.experimental.pallas import tpu_sc as plsc

VOCAB = 1000000
EMBED_DIM = 16
BATCH = 16384

_info = plsc.get_sparse_core_info()
_NC, _NS = _info.num_cores, _info.num_subcores
_NW = _NC * _NS
_B_PER_W = BATCH // _NW
_LANES = 16
_WAVE = 16
_N_WAVES = _B_PER_W // _WAVE
_NBUF = 3

_mesh = plsc.VectorSubcoreMesh(core_axis_name="c", subcore_axis_name="s")


@functools.partial(
    pl.kernel,
    mesh=_mesh,
    out_type=jax.ShapeDtypeStruct((EMBED_DIM, BATCH), jnp.float32),
    scratch_types=[
        pltpu.VMEM((_B_PER_W,), jnp.int32),
        pltpu.VMEM((_NBUF, _WAVE, EMBED_DIM, 128), jnp.float32),
        pltpu.VMEM((EMBED_DIM, _B_PER_W), jnp.float32),
        pltpu.SemaphoreType.DMA((_NBUF,)),
    ],
    compiler_params=pltpu.CompilerParams(needs_layout_passes=False),
)
def _gather_kernel(table_hbm, idx_hbm, out_hbm, idx_v, blocks_v, stage_v,
                   sems):
    wid = lax.axis_index("s") * _NC + lax.axis_index("c")
    base = wid * _B_PER_W
    pltpu.sync_copy(idx_hbm.at[pl.ds(base, _B_PER_W)], idx_v)

    lanes = lax.iota(jnp.int32, _LANES)
    zero128 = pl.multiple_of(jnp.int32(0), 128)

    def issue_wave(k, buf):
        vec = idx_v[pl.ds(k * _WAVE, _WAVE)]
        vbase_vec = (vec >> 7) << 7
        for l in range(_WAVE):
            vbase = jnp.sum(jnp.where(lanes == l, vbase_vec, 0))
            vbase = pl.multiple_of(vbase, 128)
            pltpu.async_copy(
                table_hbm.at[lanes, pl.ds(vbase, 128)],
                blocks_v.at[buf, l],
                sems.at[buf],
            )

    def drain_wave(buf):
        for l in range(_WAVE):
            pltpu.make_async_copy(
                table_hbm.at[lanes, pl.ds(zero128, 128)],
                blocks_v.at[buf, l],
                sems.at[buf],
            ).wait()

    def select_wave(k, buf):
        vec = idx_v[pl.ds(k * _WAVE, _WAVE)]
        cvec = vec & 127
        bufvec = jnp.full((_LANES,), buf, jnp.int32)
        for l in range(_WAVE):
            c = jnp.sum(jnp.where(lanes == l, cvec, 0))
            vals = plsc.load_gather(
                blocks_v,
                [
                    bufvec,
                    jnp.full((_LANES,), l, jnp.int32),
                    lanes,
                    jnp.full((_LANES,), c, jnp.int32),
                ],
            )
            plsc.store_scatter(
                stage_v,
                [lanes, jnp.full((_LANES,), k * _WAVE + l, jnp.int32)],
                vals,
            )

    for p in range(_NBUF - 1):
        issue_wave(p, jnp.int32(p))

    def wave_body(k, carry):
        buf = lax.rem(k, _NBUF)
        nxt = lax.rem(k + _NBUF - 1, _NBUF)

        @pl.when(k + _NBUF - 1 < _N_WAVES)
        def _():
            issue_wave(k + _NBUF - 1, nxt)

        drain_wave(buf)
        select_wave(k, buf)
        return carry

    lax.fori_loop(0, _N_WAVES, wave_body, 0)

    pltpu.sync_copy(stage_v, out_hbm.at[:, pl.ds(base, _B_PER_W)])


def kernel(g, x, attri):
    idx = jnp.squeeze(x).astype(jnp.int32)
    out_t = _gather_kernel(attri.T, idx)
    return out_t.T
